# baseline (speedup 1.0000x reference)
import functools

import jax
import jax.numpy as jnp
from jax import lax
from jax.experimental import pallas as pl
from jax.experimental.pallas import tpu as pltpu
from jax.experimental.pallas import tpu_sc as plsc

N = 10000
E = 320000
DIN = 128
H2 = 128
H = 64
O = 2

NC, NS, L = 2, 16, 16
NW = NC * NS
CHUNK = 128
EP = 327680
NCH = EP // CHUNK
ROWS_W = NCH // NW
G = 16
NG = ROWS_W // G
AGG_W0 = 80
AGG_W1 = 2 * ROWS_W - AGG_W0
NP = 10240
RPT = NP // NS

@functools.cache
def _mesh():
    return plsc.VectorSubcoreMesh(
        core_axis_name="c", subcore_axis_name="s",
        num_cores=NC, num_subcores=NS)


@functools.cache
def _mesh1():
    return plsc.VectorSubcoreMesh(
        core_axis_name="c", subcore_axis_name="s",
        num_cores=1, num_subcores=NS)


def _sc_degree_body(dst_hbm, zeros_hbm, ones_hbm, out_hbm, idx_v, ones_v,
                    acc_sh):
    cid = lax.axis_index("c")
    sid = lax.axis_index("s")
    wid = sid * NC + cid
    base = sid * RPT
    for k in range(RPT // CHUNK):
        pltpu.sync_copy(zeros_hbm, acc_sh.at[pl.ds(base + k * CHUNK, CHUNK)])
    pltpu.sync_copy(ones_hbm, ones_v)
    pltpu.sync_copy(dst_hbm.at[pl.ds(wid * ROWS_W, ROWS_W)], idx_v)
    plsc.subcore_barrier()

    def body(j, carry):
        pltpu.sync_copy(ones_v, acc_sh.at[idx_v.at[j]], add=True)
        return carry

    lax.fori_loop(0, ROWS_W, body, 0)
    plsc.subcore_barrier()
    pltpu.sync_copy(acc_sh.at[pl.ds(base, RPT)],
                    out_hbm.at[cid, pl.ds(base, RPT)])


def _sc_degree_zeros_ones():
    return jnp.zeros((CHUNK, 16), jnp.float32), jnp.ones((CHUNK, 16),
                                                         jnp.float32)


def _sc_degree(dst2d, zeros_row, ones_row):
    return pl.kernel(
        _sc_degree_body,
        out_type=jax.ShapeDtypeStruct((NC, NP, 16), jnp.float32),
        mesh=_mesh(),
        scratch_types=[
            pltpu.VMEM((ROWS_W, CHUNK), jnp.int32),
            pltpu.VMEM((CHUNK, 16), jnp.float32),
            pltpu.VMEM_SHARED((NP, 16), jnp.float32),
        ],
        compiler_params=pltpu.CompilerParams(use_tc_tiling_on_sc=False),
    )(dst2d, zeros_row, ones_row)


def _sc_aggregate_body(src_hbm, dst_hbm, xs_hbm, zeros_hbm, out_hbm,
                       src_r, dst_r, rows0, rows1, gsem0, gsem1, isem, zsem,
                       acc_sh):
    cid = lax.axis_index("c")
    sid = lax.axis_index("s")
    is0 = cid == 0
    nrows = jnp.where(is0, AGG_W0, AGG_W1)
    ng = nrows // G
    wbase = jnp.where(is0, sid * AGG_W0, NS * AGG_W0 + sid * AGG_W1)
    wbase = pl.multiple_of(wbase, 8)
    base = sid * RPT
    for k in range(RPT // CHUNK):
        pltpu.async_copy(zeros_hbm, acc_sh.at[pl.ds(base + k * CHUNK, CHUNK)],
                         zsem)
    pltpu.sync_copy(src_hbm.at[pl.ds(wbase, G)], src_r.at[0])
    pltpu.sync_copy(dst_hbm.at[pl.ds(wbase, G)], dst_r.at[0])

    @pl.when(ng >= 2)
    def _():
        pltpu.async_copy(src_hbm.at[pl.ds(wbase + G, G)], src_r.at[1], isem)
        pltpu.async_copy(dst_hbm.at[pl.ds(wbase + G, G)], dst_r.at[1], isem)
    for k in range(RPT // CHUNK):
        pltpu.make_async_copy(
            zeros_hbm, acc_sh.at[pl.ds(base + k * CHUNK, CHUNK)], zsem).wait()
    plsc.subcore_barrier()

    def sidx(c):
        return src_r.at[(c // G) % 2, c % G]

    def didx(c):
        return dst_r.at[(c // G) % 2, c % G]

    def _wait_idx():
        pltpu.make_async_copy(
            src_hbm.at[pl.ds(0, G)], src_r.at[0], isem).wait()
        pltpu.make_async_copy(
            src_hbm.at[pl.ds(0, G)], src_r.at[0], isem).wait()

    pltpu.async_copy(xs_hbm.at[sidx(0)], rows0, gsem0)
    half = G // 2

    def body(i, carry):
        j0 = 2 * i
        j1 = j0 + 1
        g = i // half
        at_boundary = (i % half) == (half - 1)
        pltpu.make_async_copy(xs_hbm.at[sidx(j0)], rows0, gsem0).wait()
        pltpu.async_copy(xs_hbm.at[sidx(j1)], rows1, gsem1)
        pltpu.sync_copy(rows0, acc_sh.at[didx(j0)], add=True)

        @pl.when(jnp.logical_and(at_boundary, g < ng - 1))
        def _():
            _wait_idx()

        pltpu.make_async_copy(xs_hbm.at[sidx(j1)], rows1, gsem1).wait()

        @pl.when(j0 + 2 < nrows)
        def _():
            pltpu.async_copy(xs_hbm.at[sidx(j0 + 2)], rows0, gsem0)

        pltpu.sync_copy(rows1, acc_sh.at[didx(j1)], add=True)

        @pl.when(jnp.logical_and(at_boundary, g < ng - 2))
        def _():
            off = pl.multiple_of(wbase + (g + 2) * G, 8)
            slot = g % 2
            pltpu.async_copy(src_hbm.at[pl.ds(off, G)], src_r.at[slot], isem)
            pltpu.async_copy(dst_hbm.at[pl.ds(off, G)], dst_r.at[slot], isem)

        return carry

    lax.fori_loop(0, nrows // 2, body, 0)
    plsc.subcore_barrier()
    for k in range(RPT // CHUNK):
        sl = pl.ds(base + k * CHUNK, CHUNK)
        pltpu.async_copy(acc_sh.at[sl], out_hbm.at[cid, sl], zsem)
    for k in range(RPT // CHUNK):
        sl = pl.ds(base + k * CHUNK, CHUNK)
        pltpu.make_async_copy(acc_sh.at[sl], out_hbm.at[cid, sl], zsem).wait()


def _sc_aggregate(src2d, dst2d, xs, zeros_row):
    return pl.kernel(
        _sc_aggregate_body,
        out_type=jax.ShapeDtypeStruct((NC, NP, H2), jnp.float32),
        mesh=_mesh(),
        scratch_types=[
            pltpu.VMEM((2, G, CHUNK), jnp.int32),
            pltpu.VMEM((2, G, CHUNK), jnp.int32),
            pltpu.VMEM((CHUNK, H2), jnp.float32),
            pltpu.VMEM((CHUNK, H2), jnp.float32),
            pltpu.SemaphoreType.DMA,
            pltpu.SemaphoreType.DMA,
            pltpu.SemaphoreType.DMA,
            pltpu.SemaphoreType.DMA,
            pltpu.VMEM_SHARED((NP, H2), jnp.float32),
        ],
        compiler_params=pltpu.CompilerParams(use_tc_tiling_on_sc=False),
    )(src2d, dst2d, xs, zeros_row)


def _pre_body(x_ref, wc_ref, xw_ref):
    xw_ref[...] = jnp.dot(x_ref[...], wc_ref[...],
                          preferred_element_type=jnp.float32)


def _dense_pre(xpad, W_conv):
    BR = 256
    grid = (NP // BR,)
    row = pl.BlockSpec((BR, H2), lambda i: (i, 0))
    full = pl.BlockSpec((H2, H2), lambda i: (0, 0))
    return pl.pallas_call(
        _pre_body,
        grid=grid,
        in_specs=[row, full],
        out_specs=row,
        out_shape=jax.ShapeDtypeStruct((NP, H2), jnp.float32),
    )(xpad, W_conv)


def _scale_body(d0_ref, d1_ref, xw_ref, xs_ref, dinv_ref):
    deg = d0_ref[:, 0:1] + d1_ref[:, 0:1] + 1.0
    dinv = lax.rsqrt(deg)
    dinv_ref[...] = jnp.broadcast_to(dinv, dinv_ref.shape)
    xs_ref[...] = dinv * xw_ref[...]


def _scale(d0, d1, xw):
    BR = 256
    grid = (NP // BR,)
    row16 = pl.BlockSpec((BR, 16), lambda i: (i, 0))
    row = pl.BlockSpec((BR, H2), lambda i: (i, 0))
    return pl.pallas_call(
        _scale_body,
        grid=grid,
        in_specs=[row16, row16, row],
        out_specs=[row, row16],
        out_shape=[jax.ShapeDtypeStruct((NP, H2), jnp.float32),
                   jax.ShapeDtypeStruct((NP, 16), jnp.float32)],
    )(d0, d1, xw)


def _post_body(p0_ref, p1_ref, xs_ref, dinv_ref, zt_ref, first_ref, bc_ref,
               wir_ref, bir_ref, whr_ref, bhr_ref, wiz_ref, biz_ref,
               whz_ref, bhz_ref, win_ref, bin_ref, whn_ref, bhn_ref,
               wf1_ref, bf1_ref, wf2_ref, bf2_ref, zbar_ref, out_ref):
    f32 = jnp.float32
    acc = p0_ref[...] + p1_ref[...] + xs_ref[...]
    x_emb = jax.nn.relu(dinv_ref[:, 0:1] * acc + bc_ref[...])
    zt = zt_ref[...]
    r = jax.nn.sigmoid(
        jnp.dot(x_emb, wir_ref[...], preferred_element_type=f32)
        + bir_ref[...]
        + jnp.dot(zt, whr_ref[...], preferred_element_type=f32)
        + bhr_ref[...])
    z = jax.nn.sigmoid(
        jnp.dot(x_emb, wiz_ref[...], preferred_element_type=f32)
        + biz_ref[...]
        + jnp.dot(zt, whz_ref[...], preferred_element_type=f32)
        + bhz_ref[...])
    n = jnp.tanh(
        jnp.dot(x_emb, win_ref[...], preferred_element_type=f32)
        + bin_ref[...]
        + r * (jnp.dot(zt, whn_ref[...], preferred_element_type=f32)
               + bhn_ref[...]))
    zbar_gru = (1.0 - z) * n + z * zt
    f = first_ref[0:1, 0:1]
    zbar = f * x_emb + (1.0 - f) * zbar_gru
    zbar_ref[...] = zbar
    h1 = jax.nn.relu(
        jnp.dot(zbar, wf1_ref[...], preferred_element_type=f32)
        + bf1_ref[...])
    out_ref[...] = (jnp.dot(h1, wf2_ref[...], preferred_element_type=f32)
                    + bf2_ref[...])


def _dense_post(p0, p1, xs, dinv16, ztpad, first128, b_conv,
                W_ir, b_ir, W_hr, b_hr, W_iz, b_iz, W_hz, b_hz,
                W_in_, b_in_, W_hn, b_hn, W_fc1, b_fc1, W_fc2, b_fc2):
    BR = 256
    grid = (NP // BR,)
    row = pl.BlockSpec((BR, H2), lambda i: (i, 0))
    row16 = pl.BlockSpec((BR, 16), lambda i: (i, 0))
    full = pl.BlockSpec((H2, H2), lambda i: (0, 0))
    bias = pl.BlockSpec((1, H2), lambda i: (0, 0))
    return pl.pallas_call(
        _post_body,
        grid=grid,
        in_specs=[row, row, row, row16, row,
                  pl.BlockSpec((1, H2), lambda i: (0, 0)),
                  bias, full, bias, full, bias, full, bias, full, bias,
                  full, bias, full, bias,
                  pl.BlockSpec((H2, H), lambda i: (0, 0)),
                  pl.BlockSpec((1, H), lambda i: (0, 0)),
                  pl.BlockSpec((H, O), lambda i: (0, 0)),
                  pl.BlockSpec((1, O), lambda i: (0, 0))],
        out_specs=[row, pl.BlockSpec((BR, O), lambda i: (i, 0))],
        out_shape=[jax.ShapeDtypeStruct((NP, H2), jnp.float32),
                   jax.ShapeDtypeStruct((NP, O), jnp.float32)],
    )(p0, p1, xs, dinv16, ztpad, first128, b_conv,
      W_ir, b_ir, W_hr, b_hr, W_iz, b_iz, W_hz, b_hz,
      W_in_, b_in_, W_hn, b_hn, W_fc1, b_fc1, W_fc2, b_fc2)


def kernel(x, edge_index, Zt, first, W_conv, b_conv, W_ir, b_ir, W_hr, b_hr,
           W_iz, b_iz, W_hz, b_hz, W_in_, b_in_, W_hn, b_hn,
           W_fc1, b_fc1, W_fc2, b_fc2):
    f32 = jnp.float32
    pad = N + (jnp.arange(EP - E, dtype=jnp.int32) % (NP - N))
    src2d = jnp.concatenate([edge_index[0], pad]).reshape(NCH, CHUNK)
    dst2d = jnp.concatenate([edge_index[1], pad]).reshape(NCH, CHUNK)

    xpad = jnp.zeros((NP, DIN), f32).at[:N].set(x)
    ztpad = jnp.zeros((NP, H2), f32).at[:N].set(Zt)

    zeros_row = jnp.zeros((CHUNK, H2), f32)
    first128 = jnp.broadcast_to(
        jnp.asarray(first, f32).reshape(1, 1), (1, H2))

    b2 = lambda b: b.reshape(1, -1)

    xw = _dense_pre(xpad, W_conv)
    zeros16, ones16 = _sc_degree_zeros_ones()
    deg_parts = _sc_degree(dst2d, zeros16, ones16)
    xs, dinv16 = _scale(deg_parts[0], deg_parts[1], xw)
    parts = _sc_aggregate(src2d, dst2d, xs, zeros_row)
    zbar, out2 = _dense_post(parts[0], parts[1], xs, dinv16, ztpad,
                             first128, b2(b_conv), W_ir, b2(b_ir),
                             W_hr, b2(b_hr), W_iz, b2(b_iz),
                             W_hz, b2(b_hz), W_in_, b2(b_in_),
                             W_hn, b2(b_hn),
                             W_fc1, b2(b_fc1), W_fc2, b2(b_fc2))
    return out2[:N], zbar[:N]


# speedup vs baseline: 1.0009x; 1.0009x over previous
import functools

import jax
import jax.numpy as jnp
from jax import lax
from jax.experimental import pallas as pl
from jax.experimental.pallas import tpu as pltpu
from jax.experimental.pallas import tpu_sc as plsc

N = 10000
E = 320000
DIN = 128
H2 = 128
H = 64
O = 2

NC, NS, L = 2, 16, 16
NW = NC * NS
CHUNK = 128
EP = 327680
NCH = EP // CHUNK
ROWS_W = NCH // NW
G = 16
NG = ROWS_W // G
AGG_W0 = 80
AGG_W1 = 2 * ROWS_W - AGG_W0
NP = 10240
RPT = NP // NS

@functools.cache
def _mesh():
    return plsc.VectorSubcoreMesh(
        core_axis_name="c", subcore_axis_name="s",
        num_cores=NC, num_subcores=NS)


@functools.cache
def _mesh1():
    return plsc.VectorSubcoreMesh(
        core_axis_name="c", subcore_axis_name="s",
        num_cores=1, num_subcores=NS)


def _sc_degree_body(dst_hbm, zeros_hbm, ones_hbm, out_hbm, idx_v, ones_v,
                    acc_sh):
    cid = lax.axis_index("c")
    sid = lax.axis_index("s")
    wid = sid * NC + cid
    base = sid * RPT
    for k in range(RPT // CHUNK):
        pltpu.sync_copy(zeros_hbm, acc_sh.at[pl.ds(base + k * CHUNK, CHUNK)])
    pltpu.sync_copy(ones_hbm, ones_v)
    pltpu.sync_copy(dst_hbm.at[pl.ds(wid * ROWS_W, ROWS_W)], idx_v)
    plsc.subcore_barrier()

    def body(j, carry):
        pltpu.sync_copy(ones_v, acc_sh.at[idx_v.at[j]], add=True)
        return carry

    lax.fori_loop(0, ROWS_W, body, 0)
    plsc.subcore_barrier()
    pltpu.sync_copy(acc_sh.at[pl.ds(base, RPT)],
                    out_hbm.at[cid, pl.ds(base, RPT)])


def _sc_degree_zeros_ones():
    return jnp.zeros((CHUNK, 16), jnp.float32), jnp.ones((CHUNK, 16),
                                                         jnp.float32)


def _sc_degree(dst2d, zeros_row, ones_row):
    return pl.kernel(
        _sc_degree_body,
        out_type=jax.ShapeDtypeStruct((NC, NP, 16), jnp.float32),
        mesh=_mesh(),
        scratch_types=[
            pltpu.VMEM((ROWS_W, CHUNK), jnp.int32),
            pltpu.VMEM((CHUNK, 16), jnp.float32),
            pltpu.VMEM_SHARED((NP, 16), jnp.float32),
        ],
        compiler_params=pltpu.CompilerParams(use_tc_tiling_on_sc=False),
    )(dst2d, zeros_row, ones_row)


def _sc_aggregate_body(src_hbm, dst_hbm, xs_hbm, zeros_hbm, out_hbm,
                       src_r, dst_r, rows0, rows1, gsem0, gsem1, isem, zsem,
                       acc_sh):
    cid = lax.axis_index("c")
    sid = lax.axis_index("s")
    is0 = cid == 0
    nrows = jnp.where(is0, AGG_W0, AGG_W1)
    ng = nrows // G
    wbase = jnp.where(is0, sid * AGG_W0, NS * AGG_W0 + sid * AGG_W1)
    wbase = pl.multiple_of(wbase, 8)
    base = sid * RPT
    for k in range(RPT // CHUNK):
        pltpu.async_copy(zeros_hbm, acc_sh.at[pl.ds(base + k * CHUNK, CHUNK)],
                         zsem)
    pltpu.sync_copy(src_hbm.at[pl.ds(wbase, G)], src_r.at[0])
    pltpu.sync_copy(dst_hbm.at[pl.ds(wbase, G)], dst_r.at[0])

    @pl.when(ng >= 2)
    def _():
        pltpu.async_copy(src_hbm.at[pl.ds(wbase + G, G)], src_r.at[1], isem)
        pltpu.async_copy(dst_hbm.at[pl.ds(wbase + G, G)], dst_r.at[1], isem)
    for k in range(RPT // CHUNK):
        pltpu.make_async_copy(
            zeros_hbm, acc_sh.at[pl.ds(base + k * CHUNK, CHUNK)], zsem).wait()
    plsc.subcore_barrier()

    def sidx(c):
        return src_r.at[(c // G) % 2, c % G]

    def didx(c):
        return dst_r.at[(c // G) % 2, c % G]

    def _wait_idx():
        pltpu.make_async_copy(
            src_hbm.at[pl.ds(0, G)], src_r.at[0], isem).wait()
        pltpu.make_async_copy(
            src_hbm.at[pl.ds(0, G)], src_r.at[0], isem).wait()

    pltpu.async_copy(xs_hbm.at[sidx(0)], rows0, gsem0)
    half = G // 2

    def body(i, carry):
        j0 = 2 * i
        j1 = j0 + 1
        g = i // half
        at_boundary = (i % half) == (half - 1)
        pltpu.make_async_copy(xs_hbm.at[sidx(j0)], rows0, gsem0).wait()
        pltpu.async_copy(xs_hbm.at[sidx(j1)], rows1, gsem1)
        pltpu.sync_copy(rows0, acc_sh.at[didx(j0)], add=True)

        @pl.when(jnp.logical_and(at_boundary, g < ng - 1))
        def _():
            _wait_idx()

        pltpu.make_async_copy(xs_hbm.at[sidx(j1)], rows1, gsem1).wait()

        @pl.when(j0 + 2 < nrows)
        def _():
            pltpu.async_copy(xs_hbm.at[sidx(j0 + 2)], rows0, gsem0)

        pltpu.sync_copy(rows1, acc_sh.at[didx(j1)], add=True)

        @pl.when(jnp.logical_and(at_boundary, g < ng - 2))
        def _():
            off = pl.multiple_of(wbase + (g + 2) * G, 8)
            slot = g % 2
            pltpu.async_copy(src_hbm.at[pl.ds(off, G)], src_r.at[slot], isem)
            pltpu.async_copy(dst_hbm.at[pl.ds(off, G)], dst_r.at[slot], isem)

        return carry

    lax.fori_loop(0, nrows // 2, body, 0)
    plsc.subcore_barrier()
    for k in range(RPT // CHUNK):
        sl = pl.ds(base + k * CHUNK, CHUNK)
        pltpu.async_copy(acc_sh.at[sl], out_hbm.at[cid, sl], zsem)
    for k in range(RPT // CHUNK):
        sl = pl.ds(base + k * CHUNK, CHUNK)
        pltpu.make_async_copy(acc_sh.at[sl], out_hbm.at[cid, sl], zsem).wait()


def _sc_aggregate(src2d, dst2d, xs, zeros_row):
    return pl.kernel(
        _sc_aggregate_body,
        out_type=jax.ShapeDtypeStruct((NC, NP, H2), jnp.float32),
        mesh=_mesh(),
        scratch_types=[
            pltpu.VMEM((2, G, CHUNK), jnp.int32),
            pltpu.VMEM((2, G, CHUNK), jnp.int32),
            pltpu.VMEM((CHUNK, H2), jnp.float32),
            pltpu.VMEM((CHUNK, H2), jnp.float32),
            pltpu.SemaphoreType.DMA,
            pltpu.SemaphoreType.DMA,
            pltpu.SemaphoreType.DMA,
            pltpu.SemaphoreType.DMA,
            pltpu.VMEM_SHARED((NP, H2), jnp.float32),
        ],
    )(src2d, dst2d, xs, zeros_row)


def _pre_body(x_ref, wc_ref, xw_ref):
    xw_ref[...] = jnp.dot(x_ref[...], wc_ref[...],
                          preferred_element_type=jnp.float32)


def _dense_pre(xpad, W_conv):
    BR = 256
    grid = (NP // BR,)
    row = pl.BlockSpec((BR, H2), lambda i: (i, 0))
    full = pl.BlockSpec((H2, H2), lambda i: (0, 0))
    return pl.pallas_call(
        _pre_body,
        grid=grid,
        in_specs=[row, full],
        out_specs=row,
        out_shape=jax.ShapeDtypeStruct((NP, H2), jnp.float32),
    )(xpad, W_conv)


def _scale_body(d0_ref, d1_ref, xw_ref, xs_ref, dinv_ref):
    deg = d0_ref[:, 0:1] + d1_ref[:, 0:1] + 1.0
    dinv = lax.rsqrt(deg)
    dinv_ref[...] = jnp.broadcast_to(dinv, dinv_ref.shape)
    xs_ref[...] = dinv * xw_ref[...]


def _scale(d0, d1, xw):
    BR = 256
    grid = (NP // BR,)
    row16 = pl.BlockSpec((BR, 16), lambda i: (i, 0))
    row = pl.BlockSpec((BR, H2), lambda i: (i, 0))
    return pl.pallas_call(
        _scale_body,
        grid=grid,
        in_specs=[row16, row16, row],
        out_specs=[row, row16],
        out_shape=[jax.ShapeDtypeStruct((NP, H2), jnp.float32),
                   jax.ShapeDtypeStruct((NP, 16), jnp.float32)],
    )(d0, d1, xw)


def _post_body(p0_ref, p1_ref, xs_ref, dinv_ref, zt_ref, first_ref, bc_ref,
               wir_ref, bir_ref, whr_ref, bhr_ref, wiz_ref, biz_ref,
               whz_ref, bhz_ref, win_ref, bin_ref, whn_ref, bhn_ref,
               wf1_ref, bf1_ref, wf2_ref, bf2_ref, zbar_ref, out_ref):
    f32 = jnp.float32
    acc = p0_ref[...] + p1_ref[...] + xs_ref[...]
    x_emb = jax.nn.relu(dinv_ref[:, 0:1] * acc + bc_ref[...])
    zt = zt_ref[...]
    r = jax.nn.sigmoid(
        jnp.dot(x_emb, wir_ref[...], preferred_element_type=f32)
        + bir_ref[...]
        + jnp.dot(zt, whr_ref[...], preferred_element_type=f32)
        + bhr_ref[...])
    z = jax.nn.sigmoid(
        jnp.dot(x_emb, wiz_ref[...], preferred_element_type=f32)
        + biz_ref[...]
        + jnp.dot(zt, whz_ref[...], preferred_element_type=f32)
        + bhz_ref[...])
    n = jnp.tanh(
        jnp.dot(x_emb, win_ref[...], preferred_element_type=f32)
        + bin_ref[...]
        + r * (jnp.dot(zt, whn_ref[...], preferred_element_type=f32)
               + bhn_ref[...]))
    zbar_gru = (1.0 - z) * n + z * zt
    f = first_ref[0:1, 0:1]
    zbar = f * x_emb + (1.0 - f) * zbar_gru
    zbar_ref[...] = zbar
    h1 = jax.nn.relu(
        jnp.dot(zbar, wf1_ref[...], preferred_element_type=f32)
        + bf1_ref[...])
    out_ref[...] = (jnp.dot(h1, wf2_ref[...], preferred_element_type=f32)
                    + bf2_ref[...])


def _dense_post(p0, p1, xs, dinv16, ztpad, first128, b_conv,
                W_ir, b_ir, W_hr, b_hr, W_iz, b_iz, W_hz, b_hz,
                W_in_, b_in_, W_hn, b_hn, W_fc1, b_fc1, W_fc2, b_fc2):
    BR = 256
    grid = (NP // BR,)
    row = pl.BlockSpec((BR, H2), lambda i: (i, 0))
    row16 = pl.BlockSpec((BR, 16), lambda i: (i, 0))
    full = pl.BlockSpec((H2, H2), lambda i: (0, 0))
    bias = pl.BlockSpec((1, H2), lambda i: (0, 0))
    return pl.pallas_call(
        _post_body,
        grid=grid,
        in_specs=[row, row, row, row16, row,
                  pl.BlockSpec((1, H2), lambda i: (0, 0)),
                  bias, full, bias, full, bias, full, bias, full, bias,
                  full, bias, full, bias,
                  pl.BlockSpec((H2, H), lambda i: (0, 0)),
                  pl.BlockSpec((1, H), lambda i: (0, 0)),
                  pl.BlockSpec((H, O), lambda i: (0, 0)),
                  pl.BlockSpec((1, O), lambda i: (0, 0))],
        out_specs=[row, pl.BlockSpec((BR, O), lambda i: (i, 0))],
        out_shape=[jax.ShapeDtypeStruct((NP, H2), jnp.float32),
                   jax.ShapeDtypeStruct((NP, O), jnp.float32)],
    )(p0, p1, xs, dinv16, ztpad, first128, b_conv,
      W_ir, b_ir, W_hr, b_hr, W_iz, b_iz, W_hz, b_hz,
      W_in_, b_in_, W_hn, b_hn, W_fc1, b_fc1, W_fc2, b_fc2)


def kernel(x, edge_index, Zt, first, W_conv, b_conv, W_ir, b_ir, W_hr, b_hr,
           W_iz, b_iz, W_hz, b_hz, W_in_, b_in_, W_hn, b_hn,
           W_fc1, b_fc1, W_fc2, b_fc2):
    f32 = jnp.float32
    pad = N + (jnp.arange(EP - E, dtype=jnp.int32) % (NP - N))
    src2d = jnp.concatenate([edge_index[0], pad]).reshape(NCH, CHUNK)
    dst2d = jnp.concatenate([edge_index[1], pad]).reshape(NCH, CHUNK)

    xpad = jnp.zeros((NP, DIN), f32).at[:N].set(x)
    ztpad = jnp.zeros((NP, H2), f32).at[:N].set(Zt)

    zeros_row = jnp.zeros((CHUNK, H2), f32)
    first128 = jnp.broadcast_to(
        jnp.asarray(first, f32).reshape(1, 1), (1, H2))

    b2 = lambda b: b.reshape(1, -1)

    xw = _dense_pre(xpad, W_conv)
    zeros16, ones16 = _sc_degree_zeros_ones()
    deg_parts = _sc_degree(dst2d, zeros16, ones16)
    xs, dinv16 = _scale(deg_parts[0], deg_parts[1], xw)
    parts = _sc_aggregate(src2d, dst2d, xs, zeros_row)
    zbar, out2 = _dense_post(parts[0], parts[1], xs, dinv16, ztpad,
                             first128, b2(b_conv), W_ir, b2(b_ir),
                             W_hr, b2(b_hr), W_iz, b2(b_iz),
                             W_hz, b2(b_hz), W_in_, b2(b_in_),
                             W_hn, b2(b_hn),
                             W_fc1, b2(b_fc1), W_fc2, b2(b_fc2))
    return out2[:N], zbar[:N]
